# baseline (speedup 1.0000x reference)
import jax
import jax.numpy as jnp
from jax.experimental import pallas as pl
from jax.experimental.pallas import tpu as pltpu

_BM = 80


def _body(x_ref, wx_ref, g_ref, h0_ref, wh_ref, o_ref, xw_ref):
    @pl.when(pl.program_id(0) == 0)
    def _pre():
        xw_ref[...] = jnp.dot(x_ref[...], wx_ref[...],
                              preferred_element_type=jnp.float32)

    o_ref[...] = (jnp.dot(g_ref[...], xw_ref[...],
                          preferred_element_type=jnp.float32)
                  + jnp.dot(h0_ref[...], wh_ref[...],
                            preferred_element_type=jnp.float32))


def kernel(input, adj, h0, lamda, alpha, l, G, weight):
    n, d = input.shape
    theta = jnp.log(lamda / l + 1.0)
    wp = theta * weight + (1.0 - theta) * jnp.eye(d, dtype=jnp.float32)
    wx = ((1.0 - alpha) * wp).astype(jnp.float32)
    wh = (alpha * wp).astype(jnp.float32)

    out = pl.pallas_call(
        _body,
        grid=(n // _BM,),
        in_specs=[
            pl.BlockSpec((n, d), lambda i: (0, 0)),
            pl.BlockSpec((d, d), lambda i: (0, 0)),
            pl.BlockSpec((_BM, n), lambda i: (i, 0)),
            pl.BlockSpec((_BM, d), lambda i: (i, 0)),
            pl.BlockSpec((d, d), lambda i: (0, 0)),
        ],
        out_specs=pl.BlockSpec((_BM, d), lambda i: (i, 0)),
        out_shape=jax.ShapeDtypeStruct((n, d), jnp.float32),
        scratch_shapes=[pltpu.VMEM((n, d), jnp.float32)],
        compiler_params=pltpu.CompilerParams(
            dimension_semantics=("arbitrary",)),
    )(input, wx, G, h0, wh)
    return out


# speedup vs baseline: 1.3420x; 1.3420x over previous
import jax
import jax.numpy as jnp
from jax.experimental import pallas as pl
from jax.experimental.pallas import tpu as pltpu

_BM = 200


def _body(x_ref, wx_ref, g_ref, h0_ref, wh_ref, o_ref, xw_ref):
    @pl.when(pl.program_id(0) == 0)
    def _pre():
        xw_ref[...] = jnp.dot(x_ref[...], wx_ref[...],
                              preferred_element_type=jnp.float32)

    o_ref[...] = (jnp.dot(g_ref[...], xw_ref[...],
                          preferred_element_type=jnp.float32)
                  + jnp.dot(h0_ref[...], wh_ref[...],
                            preferred_element_type=jnp.float32))


def kernel(input, adj, h0, lamda, alpha, l, G, weight):
    n, d = input.shape
    theta = jnp.log(lamda / l + 1.0)
    wp = theta * weight + (1.0 - theta) * jnp.eye(d, dtype=jnp.float32)
    wx = ((1.0 - alpha) * wp).astype(jnp.float32)
    wh = (alpha * wp).astype(jnp.float32)

    out = pl.pallas_call(
        _body,
        grid=(n // _BM,),
        in_specs=[
            pl.BlockSpec((n, d), lambda i: (0, 0)),
            pl.BlockSpec((d, d), lambda i: (0, 0)),
            pl.BlockSpec((_BM, n), lambda i: (i, 0)),
            pl.BlockSpec((_BM, d), lambda i: (i, 0)),
            pl.BlockSpec((d, d), lambda i: (0, 0)),
        ],
        out_specs=pl.BlockSpec((_BM, d), lambda i: (i, 0)),
        out_shape=jax.ShapeDtypeStruct((n, d), jnp.float32),
        scratch_shapes=[pltpu.VMEM((n, d), jnp.float32)],
        compiler_params=pltpu.CompilerParams(
            dimension_semantics=("arbitrary",)),
    )(input, wx, G, h0, wh)
    return out


# speedup vs baseline: 1.3433x; 1.0010x over previous
import jax
import jax.numpy as jnp
from jax.experimental import pallas as pl
from jax.experimental.pallas import tpu as pltpu

_BM = 200


def _body(scal_ref, w_ref, x_ref, g_ref, h0_ref, o_ref, xw_ref, wh_ref):
    @pl.when(pl.program_id(0) == 0)
    def _pre():
        d = w_ref.shape[0]
        theta = scal_ref[0]
        one_m_theta = scal_ref[1]
        one_m_alpha = scal_ref[2]
        alpha = scal_ref[3]
        rows = jax.lax.broadcasted_iota(jnp.int32, (d, d), 0)
        cols = jax.lax.broadcasted_iota(jnp.int32, (d, d), 1)
        eye = jnp.where(rows == cols, 1.0, 0.0).astype(jnp.float32)
        wp = theta * w_ref[...] + one_m_theta * eye
        wh_ref[...] = alpha * wp
        xw_ref[...] = jnp.dot(x_ref[...], one_m_alpha * wp,
                              preferred_element_type=jnp.float32)

    o_ref[...] = (jnp.dot(g_ref[...], xw_ref[...],
                          preferred_element_type=jnp.float32)
                  + jnp.dot(h0_ref[...], wh_ref[...],
                            preferred_element_type=jnp.float32))


def kernel(input, adj, h0, lamda, alpha, l, G, weight):
    n, d = input.shape
    theta = jnp.log(lamda / l + 1.0)
    scal = jnp.stack([theta, 1.0 - theta,
                      1.0 - alpha, alpha]).astype(jnp.float32)

    out = pl.pallas_call(
        _body,
        grid=(n // _BM,),
        in_specs=[
            pl.BlockSpec(memory_space=pltpu.SMEM),
            pl.BlockSpec((d, d), lambda i: (0, 0)),
            pl.BlockSpec((n, d), lambda i: (0, 0)),
            pl.BlockSpec((_BM, n), lambda i: (i, 0)),
            pl.BlockSpec((_BM, d), lambda i: (i, 0)),
        ],
        out_specs=pl.BlockSpec((_BM, d), lambda i: (i, 0)),
        out_shape=jax.ShapeDtypeStruct((n, d), jnp.float32),
        scratch_shapes=[pltpu.VMEM((n, d), jnp.float32),
                        pltpu.VMEM((d, d), jnp.float32)],
        compiler_params=pltpu.CompilerParams(
            dimension_semantics=("arbitrary",)),
    )(scal, weight, input, G, h0)
    return out
